# baseline (speedup 1.0000x reference)
import functools

import jax
import jax.numpy as jnp
from jax import lax
from jax.experimental import pallas as pl
from jax.experimental.pallas import tpu as pltpu
from jax.experimental.pallas import tpu_sc as plsc

B = 16384
D = 16
NROWS = 1000000
_NC = 2
_NS = 16
_NW = _NC * _NS
_BPW = B // _NW
_CHUNK = 64
_NCHUNK = _BPW // _CHUNK
_KG = _CHUNK // 16

_NP = D // 2
_NB = (NROWS + 1023) // 1024
_FPAD = _NB * 1024
_FROWS = _FPAD * _NP // 16
_JC = [j * (_FPAD // 16) for j in range(_NP)]

_mesh = plsc.VectorSubcoreMesh(core_axis_name="c", subcore_axis_name="s")


def _burst_base(iv):
    return iv >> 4


def _make_gather(fuse_mul):
    n_out = 1 if fuse_mul else 2

    @functools.partial(
        pl.kernel,
        mesh=_mesh,
        compiler_params=pltpu.CompilerParams(
            use_tc_tiling_on_sc=False, needs_layout_passes=False),
        out_type=[jax.ShapeDtypeStruct((B // 8, 128), jnp.float32)] * n_out,
        scratch_types=[
            pltpu.VMEM((_BPW,), jnp.int32),
            pltpu.VMEM((_BPW,), jnp.int32),
            pltpu.VMEM((_CHUNK * _NP,), jnp.int32),
            pltpu.VMEM((_CHUNK * _NP,), jnp.int32),
            pltpu.VMEM((_CHUNK * _NP, 16), jnp.float32),
            pltpu.VMEM((_CHUNK * _NP, 16), jnp.float32),
            pltpu.VMEM((_BPW // 8, 128), jnp.float32),
            pltpu.VMEM((_BPW // 8, 128), jnp.float32),
            pltpu.SemaphoreType.DMA,
        ],
    )
    def gather(uidx_hbm, iidx_hbm, f_u, f_i, *rest):
        outs, (uidx_v, iidx_v, ib_u, ib_i, g_u, g_i, v0, v1, sem) = (
            rest[:n_out], rest[n_out:])
        wid = lax.axis_index("s") * _NC + lax.axis_index("c")
        base = wid * _BPW
        pltpu.sync_copy(uidx_hbm.at[pl.ds(base, _BPW)], uidx_v)
        pltpu.sync_copy(iidx_hbm.at[pl.ds(base, _BPW)], iidx_v)

        iota = lax.iota(jnp.int32, 16)

        def chunk_body(c, carry):
            c0 = c * _CHUNK
            for k in range(_KG):
                ivu = uidx_v[pl.ds(c0 + k * 16, 16)]
                ivi = iidx_v[pl.ds(c0 + k * 16, 16)]
                bu = _burst_base(ivu)
                bi = _burst_base(ivi)
                for j in range(_NP):
                    ib_u[pl.ds(j * _CHUNK + k * 16, 16)] = bu + _JC[j]
                    ib_i[pl.ds(j * _CHUNK + k * 16, 16)] = bi + _JC[j]
            cp1 = pltpu.async_copy(f_u.at[ib_u], g_u, sem)
            cp2 = pltpu.async_copy(f_i.at[ib_i], g_i, sem)
            cp1.wait(); cp2.wait()

            def unpack(w):
                wi = plsc.bitcast(w, jnp.int32)
                lo = plsc.bitcast(wi << 16, jnp.float32)
                hi = plsc.bitcast(wi & jnp.int32(-65536), jnp.float32)
                return lo, hi

            for k in range(_KG):
                ivu = uidx_v[pl.ds(c0 + k * 16, 16)]
                ivi = iidx_v[pl.ds(c0 + k * 16, 16)]
                lu = ivu & 15
                li = ivi & 15
                rows = c0 + k * 16 + iota
                r2 = rows >> 3
                cb = (rows & 7) << 4
                for j in range(_NP):
                    gr = j * _CHUNK + k * 16 + iota
                    u0, u1 = unpack(plsc.load_gather(g_u, [gr, lu]))
                    i0, i1 = unpack(plsc.load_gather(g_i, [gr, li]))
                    if fuse_mul:
                        plsc.store_scatter(v0, [r2, cb + j], u0 * i0)
                        plsc.store_scatter(v0, [r2, cb + (j + _NP)], u1 * i1)
                    else:
                        plsc.store_scatter(v0, [r2, cb + j], u0)
                        plsc.store_scatter(v0, [r2, cb + (j + _NP)], u1)
                        plsc.store_scatter(v1, [r2, cb + j], i0)
                        plsc.store_scatter(v1, [r2, cb + (j + _NP)], i1)
            return carry

        lax.fori_loop(0, _NCHUNK, chunk_body, 0)

        out_sl = pl.ds(wid * (_BPW // 8), _BPW // 8)
        pltpu.sync_copy(v0, outs[0].at[out_sl])
        if not fuse_mul:
            pltpu.sync_copy(v1, outs[1].at[out_sl])

    return gather


_gather_mf = _make_gather(True)
_gather_mlp = _make_gather(False)


_WB = 196
_LW = _WB * 1024


def _rne_bf16_bits(x):
    u = jax.lax.bitcast_convert_type(x, jnp.int32)
    return jax.lax.shift_right_logical(u + jnp.int32(0x8000), 16)


def _lin_body(in_ref, out_ref):
    ra = _rne_bf16_bits(in_ref[0, :_NP, :])
    rb = _rne_bf16_bits(in_ref[0, _NP:, :])
    w = jax.lax.bitcast_convert_type((rb << 16) | ra, jnp.float32)
    out_ref[...] = w.reshape(out_ref.shape)


def _linearize(embT3):
    return pl.pallas_call(
        _lin_body,
        grid=(pl.cdiv(_NB, _WB),),
        in_specs=[pl.BlockSpec((1, D, _LW), lambda b: (0, 0, b))],
        out_specs=pl.BlockSpec((_NP, _WB, 8, 128), lambda b: (0, b, 0, 0)),
        out_shape=jax.ShapeDtypeStruct((_NP, _NB, 8, 128), jnp.float32),
    )(embT3)


_BLK8 = 512


def _mlp_body(pred, umlp, imlp, w1a, w1b, b1, w2, b2, woa, wob, bo, out):
    h = jnp.dot(umlp[...], w1a[...], preferred_element_type=jnp.float32)
    h = h + jnp.dot(imlp[...], w1b[...], preferred_element_type=jnp.float32)
    h = jnp.maximum(h + b1[...], 0.0)
    h = jnp.dot(h, w2[...], preferred_element_type=jnp.float32) + b2[...]
    h = jnp.maximum(h, 0.0)
    r = jnp.dot(pred[...], woa[...], preferred_element_type=jnp.float32)
    r = r + jnp.dot(h, wob[...], preferred_element_type=jnp.float32)
    out[...] = r + bo[...]


def _mlp(pred, umlp, imlp, w1a, w1b, b1, w2, b2, woa, wob, bo):
    row = pl.BlockSpec((_BLK8, 128), lambda i: (i, 0))

    def full(a):
        return pl.BlockSpec(a.shape, lambda i: (0,) * a.ndim)

    return pl.pallas_call(
        _mlp_body,
        grid=(B // 8 // _BLK8,),
        in_specs=[row, row, row,
                  full(w1a), full(w1b), full(b1), full(w2), full(b2),
                  full(woa), full(wob), full(bo)],
        out_specs=pl.BlockSpec((_BLK8, 8), lambda i: (i, 0)),
        out_shape=jax.ShapeDtypeStruct((B // 8, 8), jnp.float32),
    )(pred, umlp, imlp, w1a, w1b, b1, w2, b2, woa, wob, bo)


def kernel(user_indices, item_indices, emb_user_mf, emb_item_mf,
           emb_user_mlp, emb_item_mlp, W1, b1, W2, b2, Wout, bout):
    uidx = user_indices.astype(jnp.int32)
    iidx = item_indices.astype(jnp.int32)
    def lin(emb):
        return _linearize(emb.T.reshape(1, D, NROWS)).reshape(_FROWS, D)

    fu_mf = lin(emb_user_mf)
    fi_mf = lin(emb_item_mf)
    (pred,) = _gather_mf(uidx, iidx, fu_mf, fi_mf)
    fu_mlp = lin(emb_user_mlp)
    fi_mlp = lin(emb_item_mlp)
    umlp, imlp = _gather_mlp(uidx, iidx, fu_mlp, fi_mlp)
    eye8 = jnp.eye(8, dtype=jnp.float32)
    w1a = jnp.kron(eye8, W1[:, :D].T)
    w1b = jnp.kron(eye8, W1[:, D:].T)
    w2d = jnp.kron(eye8, W2.T)
    woa = jnp.kron(eye8, Wout[:, :D].T)
    wob = jnp.kron(eye8, Wout[:, D:].T)
    b1t = jnp.tile(b1, 8).reshape(1, -1)
    b2t = jnp.tile(b2, 8).reshape(1, -1)
    out8 = _mlp(pred, umlp, imlp, w1a, w1b, b1t, w2d, b2t, woa, wob,
                bout.reshape(1, 1))
    return out8.reshape(B, 1)


# speedup vs baseline: 1.0322x; 1.0322x over previous
import functools

import jax
import jax.numpy as jnp
from jax import lax
from jax.experimental import pallas as pl
from jax.experimental.pallas import tpu as pltpu
from jax.experimental.pallas import tpu_sc as plsc

B = 16384
D = 16
NROWS = 1000000
_NC = 2
_NS = 16
_NW = _NC * _NS
_BPW = B // _NW
_CHUNK = 64
_NCHUNK = _BPW // _CHUNK
_KG = _CHUNK // 16

_NP = D // 2
_NB = (NROWS + 1023) // 1024
_FPAD = _NB * 1024
_FROWS = _FPAD * _NP // 16
_JC = [j * (_FPAD // 16) for j in range(_NP)]

_mesh = plsc.VectorSubcoreMesh(core_axis_name="c", subcore_axis_name="s")


def _burst_base(iv):
    return iv >> 4


def _make_gather(fuse_mul):
    n_out = 1 if fuse_mul else 2

    @functools.partial(
        pl.kernel,
        mesh=_mesh,
        compiler_params=pltpu.CompilerParams(
            use_tc_tiling_on_sc=False, needs_layout_passes=False),
        out_type=[jax.ShapeDtypeStruct((B // 8, 128), jnp.float32)] * n_out,
        scratch_types=[
            pltpu.VMEM((_BPW,), jnp.int32),
            pltpu.VMEM((_BPW,), jnp.int32),
            pltpu.VMEM((2, _CHUNK * _NP), jnp.int32),
            pltpu.VMEM((2, _CHUNK * _NP), jnp.int32),
            pltpu.VMEM((2, _CHUNK * _NP, 16), jnp.float32),
            pltpu.VMEM((2, _CHUNK * _NP, 16), jnp.float32),
            pltpu.VMEM((_BPW // 8, 128), jnp.float32),
            pltpu.VMEM((_BPW // 8, 128), jnp.float32),
            pltpu.SemaphoreType.DMA,
        ],
    )
    def gather(uidx_hbm, iidx_hbm, f_u, f_i, *rest):
        outs, (uidx_v, iidx_v, ib_u, ib_i, g_u, g_i, v0, v1, sem) = (
            rest[:n_out], rest[n_out:])
        wid = lax.axis_index("s") * _NC + lax.axis_index("c")
        base = wid * _BPW
        pltpu.sync_copy(uidx_hbm.at[pl.ds(base, _BPW)], uidx_v)
        pltpu.sync_copy(iidx_hbm.at[pl.ds(base, _BPW)], iidx_v)

        iota = lax.iota(jnp.int32, 16)

        def unpack(w):
            wi = plsc.bitcast(w, jnp.int32)
            lo = plsc.bitcast(wi << 16, jnp.float32)
            hi = plsc.bitcast(wi & jnp.int32(-65536), jnp.float32)
            return lo, hi

        def build_and_fire(c):
            p = c % 2
            c0 = c * _CHUNK
            for k in range(_KG):
                ivu = uidx_v[pl.ds(c0 + k * 16, 16)]
                ivi = iidx_v[pl.ds(c0 + k * 16, 16)]
                bu = _burst_base(ivu)
                bi = _burst_base(ivi)
                for j in range(_NP):
                    ib_u[p, pl.ds(j * _CHUNK + k * 16, 16)] = bu + _JC[j]
                    ib_i[p, pl.ds(j * _CHUNK + k * 16, 16)] = bi + _JC[j]
            return (pltpu.async_copy(f_u.at[ib_u.at[p]], g_u.at[p], sem),
                    pltpu.async_copy(f_i.at[ib_i.at[p]], g_i.at[p], sem))

        def extract(c):
            p = c % 2
            c0 = c * _CHUNK
            for k in range(_KG):
                ivu = uidx_v[pl.ds(c0 + k * 16, 16)]
                ivi = iidx_v[pl.ds(c0 + k * 16, 16)]
                lu = ivu & 15
                li = ivi & 15
                rows = c0 + k * 16 + iota
                r2 = rows >> 3
                cb = (rows & 7) << 4
                for j in range(_NP):
                    gr = j * _CHUNK + k * 16 + iota
                    u0, u1 = unpack(plsc.load_gather(g_u.at[p], [gr, lu]))
                    i0, i1 = unpack(plsc.load_gather(g_i.at[p], [gr, li]))
                    if fuse_mul:
                        plsc.store_scatter(v0, [r2, cb + j], u0 * i0)
                        plsc.store_scatter(v0, [r2, cb + (j + _NP)], u1 * i1)
                    else:
                        plsc.store_scatter(v0, [r2, cb + j], u0)
                        plsc.store_scatter(v0, [r2, cb + (j + _NP)], u1)
                        plsc.store_scatter(v1, [r2, cb + j], i0)
                        plsc.store_scatter(v1, [r2, cb + (j + _NP)], i1)

        cps = build_and_fire(0)
        for c in range(1, _NCHUNK):
            nxt = build_and_fire(c)
            cps[0].wait(); cps[1].wait()
            extract(c - 1)
            cps = nxt
        cps[0].wait(); cps[1].wait()
        extract(_NCHUNK - 1)

        out_sl = pl.ds(wid * (_BPW // 8), _BPW // 8)
        pltpu.sync_copy(v0, outs[0].at[out_sl])
        if not fuse_mul:
            pltpu.sync_copy(v1, outs[1].at[out_sl])

    return gather


_gather_mf = _make_gather(True)
_gather_mlp = _make_gather(False)


_WB = 128
_LW = _WB * 1024


def _rne_bf16_bits(x):
    u = jax.lax.bitcast_convert_type(x, jnp.int32)
    return jax.lax.shift_right_logical(u + jnp.int32(0x8000), 16)


def _lin_body(in_ref, out_ref):
    ra = _rne_bf16_bits(in_ref[0, :_NP, :])
    rb = _rne_bf16_bits(in_ref[0, _NP:, :])
    w = jax.lax.bitcast_convert_type((rb << 16) | ra, jnp.float32)
    out_ref[...] = w.reshape(out_ref.shape)


def _linearize(embT3):
    return pl.pallas_call(
        _lin_body,
        grid=(pl.cdiv(_NB, _WB),),
        in_specs=[pl.BlockSpec((1, D, _LW), lambda b: (0, 0, b))],
        out_specs=pl.BlockSpec((_NP, _WB, 8, 128), lambda b: (0, b, 0, 0)),
        out_shape=jax.ShapeDtypeStruct((_NP, _NB, 8, 128), jnp.float32),
    )(embT3)


_BLK8 = 512


def _mlp_body(pred, umlp, imlp, w1a, w1b, b1, w2, b2, woa, wob, bo, out):
    h = jnp.dot(umlp[...], w1a[...], preferred_element_type=jnp.float32)
    h = h + jnp.dot(imlp[...], w1b[...], preferred_element_type=jnp.float32)
    h = jnp.maximum(h + b1[...], 0.0)
    h = jnp.dot(h, w2[...], preferred_element_type=jnp.float32) + b2[...]
    h = jnp.maximum(h, 0.0)
    r = jnp.dot(pred[...], woa[...], preferred_element_type=jnp.float32)
    r = r + jnp.dot(h, wob[...], preferred_element_type=jnp.float32)
    out[...] = r + bo[...]


def _mlp(pred, umlp, imlp, w1a, w1b, b1, w2, b2, woa, wob, bo):
    row = pl.BlockSpec((_BLK8, 128), lambda i: (i, 0))

    def full(a):
        return pl.BlockSpec(a.shape, lambda i: (0,) * a.ndim)

    return pl.pallas_call(
        _mlp_body,
        grid=(B // 8 // _BLK8,),
        in_specs=[row, row, row,
                  full(w1a), full(w1b), full(b1), full(w2), full(b2),
                  full(woa), full(wob), full(bo)],
        out_specs=pl.BlockSpec((_BLK8, 8), lambda i: (i, 0)),
        out_shape=jax.ShapeDtypeStruct((B // 8, 8), jnp.float32),
    )(pred, umlp, imlp, w1a, w1b, b1, w2, b2, woa, wob, bo)


def kernel(user_indices, item_indices, emb_user_mf, emb_item_mf,
           emb_user_mlp, emb_item_mlp, W1, b1, W2, b2, Wout, bout):
    uidx = user_indices.astype(jnp.int32)
    iidx = item_indices.astype(jnp.int32)
    def lin(emb):
        return _linearize(emb.T.reshape(1, D, NROWS)).reshape(_FROWS, D)

    fu_mf = lin(emb_user_mf)
    fi_mf = lin(emb_item_mf)
    (pred,) = _gather_mf(uidx, iidx, fu_mf, fi_mf)
    fu_mlp = lin(emb_user_mlp)
    fi_mlp = lin(emb_item_mlp)
    umlp, imlp = _gather_mlp(uidx, iidx, fu_mlp, fi_mlp)
    eye8 = jnp.eye(8, dtype=jnp.float32)
    w1a = jnp.kron(eye8, W1[:, :D].T)
    w1b = jnp.kron(eye8, W1[:, D:].T)
    w2d = jnp.kron(eye8, W2.T)
    woa = jnp.kron(eye8, Wout[:, :D].T)
    wob = jnp.kron(eye8, Wout[:, D:].T)
    b1t = jnp.tile(b1, 8).reshape(1, -1)
    b2t = jnp.tile(b2, 8).reshape(1, -1)
    out8 = _mlp(pred, umlp, imlp, w1a, w1b, b1t, w2d, b2t, woa, wob,
                bout.reshape(1, 1))
    return out8.reshape(B, 1)
